# baseline (speedup 1.0000x reference)
import functools
import math

import jax
import jax.numpy as jnp
from jax import lax
from jax.experimental import pallas as pl
from jax.experimental.pallas import tpu as pltpu
from jax.experimental.pallas import tpu_sc as plsc

LOG2PI = math.log(2.0 * math.pi)

K_TC = 0
K_SC = 1024

NC, NS, LANES = 2, 16, 16
NW = NC * NS


def _prep_body(mu_ref, lv_ref, w_ref, a_ref, b_ref, c_ref):
    mu = mu_ref[...]
    lv = lv_ref[...]
    w = w_ref[...]
    wm = jnp.max(w)
    lw = w - (wm + jnp.log(jnp.sum(jnp.exp(w - wm))))
    ev = jnp.exp(-lv)
    a_ref[...] = -0.5 * ev
    b_ref[...] = ev * mu
    c_ref[...] = -0.5 * (LOG2PI + lv + ev * mu * mu) + lw


def _tc_body(z_ref, a_ref, b_ref, c_ref, m_ref, s_ref, *, L, K, KC, final):
    nchunk = K // KC
    for l in range(L):
        zl = z_ref[:, l : l + 1]
        z2l = zl * zl
        a = a_ref[l : l + 1, :]
        b = b_ref[l : l + 1, :]
        c = c_ref[l : l + 1, :]
        macc = None
        for i in range(nchunk):
            sl = slice(i * KC, (i + 1) * KC)
            t = z2l * a[:, sl] + zl * b[:, sl] + c[:, sl]
            macc = t if macc is None else jnp.maximum(macc, t)
        m_l = jnp.max(macc, axis=1, keepdims=True)
        sacc = None
        for i in range(nchunk):
            sl = slice(i * KC, (i + 1) * KC)
            t = z2l * a[:, sl] + zl * b[:, sl] + c[:, sl]
            e = jnp.exp(t - m_l)
            sacc = e if sacc is None else sacc + e
        s_l = jnp.sum(sacc, axis=1, keepdims=True)
        if final:
            m_ref[:, l : l + 1] = m_l + jnp.log(s_l)
        else:
            m_ref[:, l : l + 1] = m_l
            s_ref[:, l : l + 1] = s_l


def _sc_body(a_hbm, b_hbm, c_hbm, z_hbm, m_hbm, s_hbm,
             a_v, b_v, c_v, z_v, m_v, s_v, *, L, K, BW):
    nch = BW // LANES
    wid = lax.axis_index("s") * NC + lax.axis_index("c")
    pltpu.sync_copy(a_hbm, a_v)
    pltpu.sync_copy(b_hbm, b_v)
    pltpu.sync_copy(c_hbm, c_v)
    pltpu.sync_copy(z_hbm.at[wid], z_v)

    def per_l(l, carry):
        zc = [z_v[pl.ds(l * BW + c * LANES, LANES)] for c in range(nch)]
        z2c = [zz * zz for zz in zc]

        def p1(j, m8):
            base = l * K + j * LANES
            a16 = a_v[pl.ds(base, LANES)]
            b16 = b_v[pl.ds(base, LANES)]
            c16 = c_v[pl.ds(base, LANES)]
            for i in range(LANES):
                a_s = a16[i]
                b_s = b16[i]
                c_s = c16[i]
                m8 = tuple(
                    jnp.maximum(m8[c], a_s * z2c[c] + b_s * zc[c] + c_s)
                    for c in range(nch)
                )
            return m8

        m8 = lax.fori_loop(
            0, K // LANES, p1,
            tuple(jnp.full((LANES,), -3.0e38, jnp.float32) for _ in range(nch)),
        )

        def p2(j, s8):
            base = l * K + j * LANES
            a16 = a_v[pl.ds(base, LANES)]
            b16 = b_v[pl.ds(base, LANES)]
            c16 = c_v[pl.ds(base, LANES)]
            for i in range(LANES):
                a_s = a16[i]
                b_s = b16[i]
                c_s = c16[i]
                s8 = tuple(
                    s8[c] + jnp.exp(a_s * z2c[c] + b_s * zc[c] + (c_s - m8[c]))
                    for c in range(nch)
                )
            return s8

        s8 = lax.fori_loop(
            0, K // LANES, p2,
            tuple(jnp.zeros((LANES,), jnp.float32) for _ in range(nch)),
        )
        for c in range(nch):
            m_v[pl.ds(l * BW + c * LANES, LANES)] = m8[c]
            s_v[pl.ds(l * BW + c * LANES, LANES)] = s8[c]
        return carry

    lax.fori_loop(0, L, per_l, 0)
    pltpu.sync_copy(m_v, m_hbm.at[wid])
    pltpu.sync_copy(s_v, s_hbm.at[wid])


def _merge_body(m1_ref, s1_ref, m2_ref, s2_ref, o_ref):
    m1 = m1_ref[...]
    s1 = s1_ref[...]
    m2 = m2_ref[...]
    s2 = s2_ref[...]
    m = jnp.maximum(m1, m2)
    o_ref[...] = m + jnp.log(s1 * jnp.exp(m1 - m) + s2 * jnp.exp(m2 - m))


def _run_sc(z, a_t, b_t, c_t, B, L):
    BW = B // NW
    a_f = a_t[:, K_TC:].reshape(L * K_SC)
    b_f = b_t[:, K_TC:].reshape(L * K_SC)
    c_f = c_t[:, K_TC:].reshape(L * K_SC)
    z_w = z.T.reshape(L, NW, BW).transpose(1, 0, 2).reshape(NW, L * BW)

    mesh = plsc.VectorSubcoreMesh(core_axis_name="c", subcore_axis_name="s")
    out_ty = [jax.ShapeDtypeStruct((NW, L * BW), jnp.float32)] * 2
    m_w, s_w = pl.kernel(
        functools.partial(_sc_body, L=L, K=K_SC, BW=BW),
        out_type=out_ty,
        mesh=mesh,
        scratch_types=[
            pltpu.VMEM((L * K_SC,), jnp.float32),
            pltpu.VMEM((L * K_SC,), jnp.float32),
            pltpu.VMEM((L * K_SC,), jnp.float32),
            pltpu.VMEM((L * BW,), jnp.float32),
            pltpu.VMEM((L * BW,), jnp.float32),
            pltpu.VMEM((L * BW,), jnp.float32),
        ],
    )(a_f, b_f, c_f, z_w)
    m = m_w.reshape(NW, L, BW).transpose(0, 2, 1).reshape(B, L)
    s = s_w.reshape(NW, L, BW).transpose(0, 2, 1).reshape(B, L)
    return m, s


def _run_tc(z, a_t, b_t, c_t, B, L, final):
    BT = 128
    KC = 128 if K_TC % 128 == 0 else K_TC
    a = a_t[:, :K_TC]
    b = b_t[:, :K_TC]
    c = c_t[:, :K_TC]
    n_out = 1 if final else 2
    outs = pl.pallas_call(
        functools.partial(_tc_body, L=L, K=K_TC, KC=KC, final=final),
        grid=(B // BT,),
        in_specs=[
            pl.BlockSpec((BT, L), lambda i: (i, 0)),
            pl.BlockSpec((L, K_TC), lambda i: (0, 0)),
            pl.BlockSpec((L, K_TC), lambda i: (0, 0)),
            pl.BlockSpec((L, K_TC), lambda i: (0, 0)),
        ],
        out_specs=[pl.BlockSpec((BT, L), lambda i: (i, 0))] * n_out,
        out_shape=[jax.ShapeDtypeStruct((B, L), jnp.float32)] * n_out,
    )(z, a, b, c)
    return outs


def _run_merge(m1, s1, m2, s2, B, L):
    BT = 512
    return pl.pallas_call(
        _merge_body,
        grid=(B // BT,),
        in_specs=[pl.BlockSpec((BT, L), lambda i: (i, 0))] * 4,
        out_specs=pl.BlockSpec((BT, L), lambda i: (i, 0)),
        out_shape=jax.ShapeDtypeStruct((B, L), jnp.float32),
    )(m1, s1, m2, s2)


def _final_log_body(m_ref, s_ref, o_ref):
    o_ref[...] = m_ref[...] + jnp.log(s_ref[...])


def kernel(z, means, logvars, w):
    B, L = z.shape
    K = means.shape[0]
    mu_t = means.T
    lv_t = logvars.T
    w2 = w.reshape(1, K)

    a_t, b_t, c_t = pl.pallas_call(
        _prep_body,
        out_shape=[jax.ShapeDtypeStruct((L, K), jnp.float32)] * 3,
    )(mu_t, lv_t, w2)

    if K_SC == 0:
        (out,) = _run_tc(z, a_t, b_t, c_t, B, L, final=True)
        return out
    m2, s2 = _run_sc(z, a_t, b_t, c_t, B, L)
    if K_TC == 0:
        BT = 512
        return pl.pallas_call(
            _final_log_body,
            grid=(B // BT,),
            in_specs=[pl.BlockSpec((BT, L), lambda i: (i, 0))] * 2,
            out_specs=pl.BlockSpec((BT, L), lambda i: (i, 0)),
            out_shape=jax.ShapeDtypeStruct((B, L), jnp.float32),
        )(m2, s2)
    m1, s1 = _run_tc(z, a_t, b_t, c_t, B, L, final=False)
    return _run_merge(m1, s1, m2, s2, B, L)


# speedup vs baseline: 3.9615x; 3.9615x over previous
import functools
import math

import jax
import jax.numpy as jnp
from jax import lax
from jax.experimental import pallas as pl
from jax.experimental.pallas import tpu as pltpu
from jax.experimental.pallas import tpu_sc as plsc

LOG2PI = math.log(2.0 * math.pi)

K_TC = 0
K_SC = 1024

NC, NS, LANES = 2, 16, 16
NW = NC * NS


def _prep_body(mu_ref, lv_ref, w_ref, a_ref, b_ref, c_ref):
    mu = mu_ref[...]
    lv = lv_ref[...]
    w = w_ref[...]
    wm = jnp.max(w)
    lw = w - (wm + jnp.log(jnp.sum(jnp.exp(w - wm))))
    ev = jnp.exp(-lv)
    a_ref[...] = -0.5 * ev
    b_ref[...] = ev * mu
    c_ref[...] = -0.5 * (LOG2PI + lv + ev * mu * mu) + lw


def _tc_body(z_ref, a_ref, b_ref, c_ref, m_ref, s_ref, *, L, K, KC, final):
    nchunk = K // KC
    for l in range(L):
        zl = z_ref[:, l : l + 1]
        z2l = zl * zl
        a = a_ref[l : l + 1, :]
        b = b_ref[l : l + 1, :]
        c = c_ref[l : l + 1, :]
        macc = None
        for i in range(nchunk):
            sl = slice(i * KC, (i + 1) * KC)
            t = z2l * a[:, sl] + zl * b[:, sl] + c[:, sl]
            macc = t if macc is None else jnp.maximum(macc, t)
        m_l = jnp.max(macc, axis=1, keepdims=True)
        sacc = None
        for i in range(nchunk):
            sl = slice(i * KC, (i + 1) * KC)
            t = z2l * a[:, sl] + zl * b[:, sl] + c[:, sl]
            e = jnp.exp(t - m_l)
            sacc = e if sacc is None else sacc + e
        s_l = jnp.sum(sacc, axis=1, keepdims=True)
        if final:
            m_ref[:, l : l + 1] = m_l + jnp.log(s_l)
        else:
            m_ref[:, l : l + 1] = m_l
            s_ref[:, l : l + 1] = s_l


def _lane_reduce(x, lane, op):
    for shift in (8, 4, 2, 1):
        idx = jnp.bitwise_xor(lane, shift)
        x = op(x, x.at[idx].get(mode="promise_in_bounds"))
    return x


def _sc_body(a_hbm, b_hbm, c_hbm, z_hbm, m_hbm, s_hbm,
             a_v, b_v, c_v, z_v, m_v, s_v, t0, t1, t2, t3, *, B, K, NP):
    wid = lax.axis_index("s") * NC + lax.axis_index("c")
    pltpu.sync_copy(a_hbm.at[wid], a_v)
    pltpu.sync_copy(b_hbm.at[wid], b_v)
    pltpu.sync_copy(c_hbm.at[wid], c_v)
    pltpu.sync_copy(z_hbm.at[wid], z_v)
    t_v = (t0, t1, t2, t3)
    nkc = K // LANES
    lane = lax.broadcasted_iota(jnp.int32, (LANES,), 0)
    neg = jnp.full((LANES,), -3.0e38, jnp.float32)
    zero = jnp.zeros((LANES,), jnp.float32)

    def per_chunk(j, carry):
        zc16 = z_v[pl.ds(j * LANES, LANES)]
        m_out = zero
        s_out = zero
        for blk in range(LANES // NP):
            z_s = [zc16[blk * NP + i] for i in range(NP)]
            zsp = [jnp.full((LANES,), z_s[i]) for i in range(NP)]
            z2sp = [zsp[i] * zsp[i] for i in range(NP)]

            def p1(q, macc):
                sl = pl.ds(q * LANES, LANES)
                a16 = a_v[sl]
                b16 = b_v[sl]
                c16 = c_v[sl]
                new = []
                for i in range(NP):
                    t = a16 * z2sp[i] + b16 * zsp[i] + c16
                    t_v[i][sl] = t
                    new.append(jnp.maximum(macc[i], t))
                return tuple(new)

            macc = lax.fori_loop(0, nkc, p1, (neg,) * NP)
            msp = [_lane_reduce(macc[i], lane, jnp.maximum) for i in range(NP)]

            def p2(q, sacc):
                sl = pl.ds(q * LANES, LANES)
                return tuple(
                    sacc[i] + jnp.exp(t_v[i][sl] - msp[i]) for i in range(NP)
                )

            sacc = lax.fori_loop(0, nkc, p2, (zero,) * NP)
            for i in range(NP):
                idx = blk * NP + i
                ssp = _lane_reduce(sacc[i], lane, jnp.add)
                m_out = jnp.where(lane == idx, msp[i], m_out)
                s_out = jnp.where(lane == idx, ssp, s_out)
        m_v[pl.ds(j * LANES, LANES)] = m_out
        s_v[pl.ds(j * LANES, LANES)] = s_out
        return carry

    lax.fori_loop(0, B // LANES, per_chunk, 0)
    pltpu.sync_copy(m_v, m_hbm.at[wid])
    pltpu.sync_copy(s_v, s_hbm.at[wid])


def _merge_body(m1_ref, s1_ref, m2_ref, s2_ref, o_ref):
    m1 = m1_ref[...]
    s1 = s1_ref[...]
    m2 = m2_ref[...]
    s2 = s2_ref[...]
    m = jnp.maximum(m1, m2)
    o_ref[...] = m + jnp.log(s1 * jnp.exp(m1 - m) + s2 * jnp.exp(m2 - m))


def _run_sc(z, a_t, b_t, c_t, B, L):
    assert L == NW
    NP = 4
    a_sc = a_t[:, K_TC:]
    b_sc = b_t[:, K_TC:]
    c_sc = c_t[:, K_TC:]
    z_t = z.T

    mesh = plsc.VectorSubcoreMesh(core_axis_name="c", subcore_axis_name="s")
    out_ty = [jax.ShapeDtypeStruct((L, B), jnp.float32)] * 2
    m_t, s_t = pl.kernel(
        functools.partial(_sc_body, B=B, K=K_SC, NP=NP),
        out_type=out_ty,
        mesh=mesh,
        scratch_types=[
            pltpu.VMEM((K_SC,), jnp.float32),
            pltpu.VMEM((K_SC,), jnp.float32),
            pltpu.VMEM((K_SC,), jnp.float32),
            pltpu.VMEM((B,), jnp.float32),
            pltpu.VMEM((B,), jnp.float32),
            pltpu.VMEM((B,), jnp.float32),
            pltpu.VMEM((K_SC,), jnp.float32),
            pltpu.VMEM((K_SC,), jnp.float32),
            pltpu.VMEM((K_SC,), jnp.float32),
            pltpu.VMEM((K_SC,), jnp.float32),
        ],
    )(a_sc, b_sc, c_sc, z_t)
    return m_t.T, s_t.T


def _run_tc(z, a_t, b_t, c_t, B, L, final):
    BT = 128
    KC = 128 if K_TC % 128 == 0 else K_TC
    a = a_t[:, :K_TC]
    b = b_t[:, :K_TC]
    c = c_t[:, :K_TC]
    n_out = 1 if final else 2
    outs = pl.pallas_call(
        functools.partial(_tc_body, L=L, K=K_TC, KC=KC, final=final),
        grid=(B // BT,),
        in_specs=[
            pl.BlockSpec((BT, L), lambda i: (i, 0)),
            pl.BlockSpec((L, K_TC), lambda i: (0, 0)),
            pl.BlockSpec((L, K_TC), lambda i: (0, 0)),
            pl.BlockSpec((L, K_TC), lambda i: (0, 0)),
        ],
        out_specs=[pl.BlockSpec((BT, L), lambda i: (i, 0))] * n_out,
        out_shape=[jax.ShapeDtypeStruct((B, L), jnp.float32)] * n_out,
    )(z, a, b, c)
    return outs


def _run_merge(m1, s1, m2, s2, B, L):
    BT = 512
    return pl.pallas_call(
        _merge_body,
        grid=(B // BT,),
        in_specs=[pl.BlockSpec((BT, L), lambda i: (i, 0))] * 4,
        out_specs=pl.BlockSpec((BT, L), lambda i: (i, 0)),
        out_shape=jax.ShapeDtypeStruct((B, L), jnp.float32),
    )(m1, s1, m2, s2)


def _final_log_body(m_ref, s_ref, o_ref):
    o_ref[...] = m_ref[...] + jnp.log(s_ref[...])


def kernel(z, means, logvars, w):
    B, L = z.shape
    K = means.shape[0]
    mu_t = means.T
    lv_t = logvars.T
    w2 = w.reshape(1, K)

    a_t, b_t, c_t = pl.pallas_call(
        _prep_body,
        out_shape=[jax.ShapeDtypeStruct((L, K), jnp.float32)] * 3,
    )(mu_t, lv_t, w2)

    if K_SC == 0:
        (out,) = _run_tc(z, a_t, b_t, c_t, B, L, final=True)
        return out
    m2, s2 = _run_sc(z, a_t, b_t, c_t, B, L)
    if K_TC == 0:
        BT = 512
        return pl.pallas_call(
            _final_log_body,
            grid=(B // BT,),
            in_specs=[pl.BlockSpec((BT, L), lambda i: (i, 0))] * 2,
            out_specs=pl.BlockSpec((BT, L), lambda i: (i, 0)),
            out_shape=jax.ShapeDtypeStruct((B, L), jnp.float32),
        )(m2, s2)
    m1, s1 = _run_tc(z, a_t, b_t, c_t, B, L, final=False)
    return _run_merge(m1, s1, m2, s2, B, L)


# speedup vs baseline: 7.4799x; 1.8881x over previous
import functools
import math

import jax
import jax.numpy as jnp
from jax import lax
from jax.experimental import pallas as pl
from jax.experimental.pallas import tpu as pltpu
from jax.experimental.pallas import tpu_sc as plsc

LOG2PI = math.log(2.0 * math.pi)

K_TC = 704
K_SC = 320

NC, NS, LANES = 2, 16, 16
NW = NC * NS


def _prep_body(mu_ref, lv_ref, w_ref, a_ref, b_ref, c_ref):
    mu = mu_ref[...]
    lv = lv_ref[...]
    w = w_ref[...]
    wm = jnp.max(w)
    lw = w - (wm + jnp.log(jnp.sum(jnp.exp(w - wm))))
    ev = jnp.exp(-lv)
    a_ref[...] = -0.5 * ev
    b_ref[...] = ev * mu
    c_ref[...] = -0.5 * (LOG2PI + lv + ev * mu * mu) + lw


def _tc_body(z_ref, a_ref, b_ref, c_ref, m_ref, s_ref, *, L, K, KC, final):
    nchunk = K // KC
    for l in range(L):
        zl = z_ref[:, l : l + 1]
        z2l = zl * zl
        a = a_ref[l : l + 1, :]
        b = b_ref[l : l + 1, :]
        c = c_ref[l : l + 1, :]
        macc = None
        for i in range(nchunk):
            sl = slice(i * KC, (i + 1) * KC)
            t = z2l * a[:, sl] + zl * b[:, sl] + c[:, sl]
            macc = t if macc is None else jnp.maximum(macc, t)
        m_l = jnp.max(macc, axis=1, keepdims=True)
        sacc = None
        for i in range(nchunk):
            sl = slice(i * KC, (i + 1) * KC)
            t = z2l * a[:, sl] + zl * b[:, sl] + c[:, sl]
            e = jnp.exp(t - m_l)
            sacc = e if sacc is None else sacc + e
        s_l = jnp.sum(sacc, axis=1, keepdims=True)
        if final:
            m_ref[:, l : l + 1] = m_l + jnp.log(s_l)
        else:
            m_ref[:, l : l + 1] = m_l
            s_ref[:, l : l + 1] = s_l


def _lane_reduce(x, lane, op):
    for shift in (8, 4, 2, 1):
        idx = jnp.bitwise_xor(lane, shift)
        x = op(x, x.at[idx].get(mode="promise_in_bounds"))
    return x


def _sc_body(a_hbm, b_hbm, c_hbm, z_hbm, m_hbm, s_hbm,
             a_v, b_v, c_v, z_v, m_v, s_v, t0, t1, t2, t3, *, B, K, NP):
    wid = lax.axis_index("s") * NC + lax.axis_index("c")
    pltpu.sync_copy(a_hbm.at[wid], a_v)
    pltpu.sync_copy(b_hbm.at[wid], b_v)
    pltpu.sync_copy(c_hbm.at[wid], c_v)
    pltpu.sync_copy(z_hbm.at[wid], z_v)
    t_v = (t0, t1, t2, t3)
    nkc = K // LANES
    lane = lax.broadcasted_iota(jnp.int32, (LANES,), 0)
    neg = jnp.full((LANES,), -3.0e38, jnp.float32)
    zero = jnp.zeros((LANES,), jnp.float32)

    def per_chunk(j, carry):
        zc16 = z_v[pl.ds(j * LANES, LANES)]
        m_out = zero
        s_out = zero
        for blk in range(LANES // NP):
            z_s = [zc16[blk * NP + i] for i in range(NP)]
            zsp = [jnp.full((LANES,), z_s[i]) for i in range(NP)]
            z2sp = [zsp[i] * zsp[i] for i in range(NP)]

            def p1(q, macc):
                sl = pl.ds(q * LANES, LANES)
                a16 = a_v[sl]
                b16 = b_v[sl]
                c16 = c_v[sl]
                new = []
                for i in range(NP):
                    t = a16 * z2sp[i] + b16 * zsp[i] + c16
                    t_v[i][sl] = t
                    new.append(jnp.maximum(macc[i], t))
                return tuple(new)

            macc = lax.fori_loop(0, nkc, p1, (neg,) * NP)
            msp = [_lane_reduce(macc[i], lane, jnp.maximum) for i in range(NP)]

            def p2(q, sacc):
                sl = pl.ds(q * LANES, LANES)
                return tuple(
                    sacc[i] + jnp.exp(t_v[i][sl] - msp[i]) for i in range(NP)
                )

            sacc = lax.fori_loop(0, nkc, p2, (zero,) * NP)
            for i in range(NP):
                idx = blk * NP + i
                ssp = _lane_reduce(sacc[i], lane, jnp.add)
                m_out = jnp.where(lane == idx, msp[i], m_out)
                s_out = jnp.where(lane == idx, ssp, s_out)
        m_v[pl.ds(j * LANES, LANES)] = m_out
        s_v[pl.ds(j * LANES, LANES)] = s_out
        return carry

    lax.fori_loop(0, B // LANES, per_chunk, 0)
    pltpu.sync_copy(m_v, m_hbm.at[wid])
    pltpu.sync_copy(s_v, s_hbm.at[wid])


def _merge_body(m1_ref, s1_ref, m2_ref, s2_ref, o_ref):
    m1 = m1_ref[...]
    s1 = s1_ref[...]
    m2 = m2_ref[...]
    s2 = s2_ref[...]
    m = jnp.maximum(m1, m2)
    o_ref[...] = m + jnp.log(s1 * jnp.exp(m1 - m) + s2 * jnp.exp(m2 - m))


def _run_sc(z, a_t, b_t, c_t, B, L):
    assert L == NW
    NP = 4
    a_sc = a_t[:, K_TC:]
    b_sc = b_t[:, K_TC:]
    c_sc = c_t[:, K_TC:]
    z_t = z.T

    mesh = plsc.VectorSubcoreMesh(core_axis_name="c", subcore_axis_name="s")
    out_ty = [jax.ShapeDtypeStruct((L, B), jnp.float32)] * 2
    m_t, s_t = pl.kernel(
        functools.partial(_sc_body, B=B, K=K_SC, NP=NP),
        out_type=out_ty,
        mesh=mesh,
        scratch_types=[
            pltpu.VMEM((K_SC,), jnp.float32),
            pltpu.VMEM((K_SC,), jnp.float32),
            pltpu.VMEM((K_SC,), jnp.float32),
            pltpu.VMEM((B,), jnp.float32),
            pltpu.VMEM((B,), jnp.float32),
            pltpu.VMEM((B,), jnp.float32),
            pltpu.VMEM((K_SC,), jnp.float32),
            pltpu.VMEM((K_SC,), jnp.float32),
            pltpu.VMEM((K_SC,), jnp.float32),
            pltpu.VMEM((K_SC,), jnp.float32),
        ],
    )(a_sc, b_sc, c_sc, z_t)
    return m_t.T, s_t.T


def _run_tc(z, a_t, b_t, c_t, B, L, final):
    BT = 128
    KC = 128 if K_TC % 128 == 0 else K_TC
    a = a_t[:, :K_TC]
    b = b_t[:, :K_TC]
    c = c_t[:, :K_TC]
    n_out = 1 if final else 2
    outs = pl.pallas_call(
        functools.partial(_tc_body, L=L, K=K_TC, KC=KC, final=final),
        grid=(B // BT,),
        in_specs=[
            pl.BlockSpec((BT, L), lambda i: (i, 0)),
            pl.BlockSpec((L, K_TC), lambda i: (0, 0)),
            pl.BlockSpec((L, K_TC), lambda i: (0, 0)),
            pl.BlockSpec((L, K_TC), lambda i: (0, 0)),
        ],
        out_specs=[pl.BlockSpec((BT, L), lambda i: (i, 0))] * n_out,
        out_shape=[jax.ShapeDtypeStruct((B, L), jnp.float32)] * n_out,
    )(z, a, b, c)
    return outs


def _run_merge(m1, s1, m2, s2, B, L):
    BT = 512
    return pl.pallas_call(
        _merge_body,
        grid=(B // BT,),
        in_specs=[pl.BlockSpec((BT, L), lambda i: (i, 0))] * 4,
        out_specs=pl.BlockSpec((BT, L), lambda i: (i, 0)),
        out_shape=jax.ShapeDtypeStruct((B, L), jnp.float32),
    )(m1, s1, m2, s2)


def _final_log_body(m_ref, s_ref, o_ref):
    o_ref[...] = m_ref[...] + jnp.log(s_ref[...])


def kernel(z, means, logvars, w):
    B, L = z.shape
    K = means.shape[0]
    mu_t = means.T
    lv_t = logvars.T
    w2 = w.reshape(1, K)

    a_t, b_t, c_t = pl.pallas_call(
        _prep_body,
        out_shape=[jax.ShapeDtypeStruct((L, K), jnp.float32)] * 3,
    )(mu_t, lv_t, w2)

    if K_SC == 0:
        (out,) = _run_tc(z, a_t, b_t, c_t, B, L, final=True)
        return out
    m2, s2 = _run_sc(z, a_t, b_t, c_t, B, L)
    if K_TC == 0:
        BT = 512
        return pl.pallas_call(
            _final_log_body,
            grid=(B // BT,),
            in_specs=[pl.BlockSpec((BT, L), lambda i: (i, 0))] * 2,
            out_specs=pl.BlockSpec((BT, L), lambda i: (i, 0)),
            out_shape=jax.ShapeDtypeStruct((B, L), jnp.float32),
        )(m2, s2)
    m1, s1 = _run_tc(z, a_t, b_t, c_t, B, L, final=False)
    return _run_merge(m1, s1, m2, s2, B, L)


# speedup vs baseline: 8.6078x; 1.1508x over previous
import functools
import math

import jax
import jax.numpy as jnp
from jax import lax
from jax.experimental import pallas as pl
from jax.experimental.pallas import tpu as pltpu
from jax.experimental.pallas import tpu_sc as plsc

LOG2PI = math.log(2.0 * math.pi)

K_TC = 704
K_SC = 320

NC, NS, LANES = 2, 16, 16
NW = NC * NS


def _prep_body(mu_ref, lv_ref, w_ref, a_ref, b_ref, c_ref):
    mu = mu_ref[...]
    lv = lv_ref[...]
    w = w_ref[...]
    wm = jnp.max(w)
    lw = w - (wm + jnp.log(jnp.sum(jnp.exp(w - wm))))
    ev = jnp.exp(-lv)
    a_ref[...] = -0.5 * ev
    b_ref[...] = ev * mu
    c_ref[...] = -0.5 * (LOG2PI + lv + ev * mu * mu) + lw


def _tc_body(z_ref, a_ref, b_ref, c_ref, *refs, L, K, KC, final):
    if final:
        m_ref, t_ref = refs
        s_ref = None
    else:
        m_ref, s_ref, t_ref = refs
    nchunk = K // KC
    for l in range(L):
        zl = z_ref[:, l : l + 1]
        a = a_ref[l : l + 1, :]
        b = b_ref[l : l + 1, :]
        c = c_ref[l : l + 1, :]
        macc = None
        for i in range(nchunk):
            sl = slice(i * KC, (i + 1) * KC)
            t = (a[:, sl] * zl + b[:, sl]) * zl + c[:, sl]
            t_ref[:, sl] = t
            macc = t if macc is None else jnp.maximum(macc, t)
        m_l = jnp.max(macc, axis=1, keepdims=True)
        sacc = None
        for i in range(nchunk):
            sl = slice(i * KC, (i + 1) * KC)
            e = jnp.exp(t_ref[:, sl] - m_l)
            sacc = e if sacc is None else sacc + e
        s_l = jnp.sum(sacc, axis=1, keepdims=True)
        if final:
            m_ref[:, l : l + 1] = m_l + jnp.log(s_l)
        else:
            m_ref[:, l : l + 1] = m_l
            s_ref[:, l : l + 1] = s_l


def _lane_reduce(x, lane, op):
    for shift in (8, 4, 2, 1):
        idx = jnp.bitwise_xor(lane, shift)
        x = op(x, x.at[idx].get(mode="promise_in_bounds"))
    return x


def _sc_body(a_hbm, b_hbm, c_hbm, z_hbm, m_hbm, s_hbm,
             a_v, b_v, c_v, z_v, m_v, s_v, t0, t1, t2, t3, *, B, K, NP):
    wid = lax.axis_index("s") * NC + lax.axis_index("c")
    pltpu.sync_copy(a_hbm.at[wid], a_v)
    pltpu.sync_copy(b_hbm.at[wid], b_v)
    pltpu.sync_copy(c_hbm.at[wid], c_v)
    pltpu.sync_copy(z_hbm.at[wid], z_v)
    t_v = (t0, t1, t2, t3)
    nkc = K // LANES
    lane = lax.broadcasted_iota(jnp.int32, (LANES,), 0)
    neg = jnp.full((LANES,), -3.0e38, jnp.float32)
    zero = jnp.zeros((LANES,), jnp.float32)

    def per_chunk(j, carry):
        zc16 = z_v[pl.ds(j * LANES, LANES)]
        m_out = zero
        s_out = zero
        for blk in range(LANES // NP):
            z_s = [zc16[blk * NP + i] for i in range(NP)]
            zsp = [jnp.full((LANES,), z_s[i]) for i in range(NP)]
            z2sp = [zsp[i] * zsp[i] for i in range(NP)]

            def p1(q, macc):
                sl = pl.ds(q * LANES, LANES)
                a16 = a_v[sl]
                b16 = b_v[sl]
                c16 = c_v[sl]
                new = []
                for i in range(NP):
                    t = a16 * z2sp[i] + b16 * zsp[i] + c16
                    t_v[i][sl] = t
                    new.append(jnp.maximum(macc[i], t))
                return tuple(new)

            macc = lax.fori_loop(0, nkc, p1, (neg,) * NP)
            msp = [_lane_reduce(macc[i], lane, jnp.maximum) for i in range(NP)]

            def p2(q, sacc):
                sl = pl.ds(q * LANES, LANES)
                return tuple(
                    sacc[i] + jnp.exp(t_v[i][sl] - msp[i]) for i in range(NP)
                )

            sacc = lax.fori_loop(0, nkc, p2, (zero,) * NP)
            for i in range(NP):
                idx = blk * NP + i
                ssp = _lane_reduce(sacc[i], lane, jnp.add)
                m_out = jnp.where(lane == idx, msp[i], m_out)
                s_out = jnp.where(lane == idx, ssp, s_out)
        m_v[pl.ds(j * LANES, LANES)] = m_out
        s_v[pl.ds(j * LANES, LANES)] = s_out
        return carry

    lax.fori_loop(0, B // LANES, per_chunk, 0)
    pltpu.sync_copy(m_v, m_hbm.at[wid])
    pltpu.sync_copy(s_v, s_hbm.at[wid])


def _merge_body(m1_ref, s1_ref, m2_ref, s2_ref, o_ref):
    m1 = m1_ref[...]
    s1 = s1_ref[...]
    m2 = m2_ref[...].T
    s2 = s2_ref[...].T
    m = jnp.maximum(m1, m2)
    o_ref[...] = m + jnp.log(s1 * jnp.exp(m1 - m) + s2 * jnp.exp(m2 - m))


def _run_sc(z, a_t, b_t, c_t, B, L):
    assert L == NW
    NP = 4
    a_sc = a_t[:, K_TC:]
    b_sc = b_t[:, K_TC:]
    c_sc = c_t[:, K_TC:]
    z_t = z.T

    mesh = plsc.VectorSubcoreMesh(core_axis_name="c", subcore_axis_name="s")
    out_ty = [jax.ShapeDtypeStruct((L, B), jnp.float32)] * 2
    m_t, s_t = pl.kernel(
        functools.partial(_sc_body, B=B, K=K_SC, NP=NP),
        out_type=out_ty,
        mesh=mesh,
        scratch_types=[
            pltpu.VMEM((K_SC,), jnp.float32),
            pltpu.VMEM((K_SC,), jnp.float32),
            pltpu.VMEM((K_SC,), jnp.float32),
            pltpu.VMEM((B,), jnp.float32),
            pltpu.VMEM((B,), jnp.float32),
            pltpu.VMEM((B,), jnp.float32),
            pltpu.VMEM((K_SC,), jnp.float32),
            pltpu.VMEM((K_SC,), jnp.float32),
            pltpu.VMEM((K_SC,), jnp.float32),
            pltpu.VMEM((K_SC,), jnp.float32),
        ],
    )(a_sc, b_sc, c_sc, z_t)
    return m_t, s_t


def _run_tc(z, a_t, b_t, c_t, B, L, final):
    BT = 128
    KC = 128 if K_TC % 128 == 0 else K_TC
    a = a_t[:, :K_TC]
    b = b_t[:, :K_TC]
    c = c_t[:, :K_TC]
    n_out = 1 if final else 2
    outs = pl.pallas_call(
        functools.partial(_tc_body, L=L, K=K_TC, KC=KC, final=final),
        grid=(B // BT,),
        in_specs=[
            pl.BlockSpec((BT, L), lambda i: (i, 0)),
            pl.BlockSpec((L, K_TC), lambda i: (0, 0)),
            pl.BlockSpec((L, K_TC), lambda i: (0, 0)),
            pl.BlockSpec((L, K_TC), lambda i: (0, 0)),
        ],
        out_specs=[pl.BlockSpec((BT, L), lambda i: (i, 0))] * n_out,
        out_shape=[jax.ShapeDtypeStruct((B, L), jnp.float32)] * n_out,
        scratch_shapes=[pltpu.VMEM((BT, K_TC), jnp.float32)],
    )(z, a, b, c)
    return outs


def _run_merge(m1, s1, m2_t, s2_t, B, L):
    BT = 512
    return pl.pallas_call(
        _merge_body,
        grid=(B // BT,),
        in_specs=[
            pl.BlockSpec((BT, L), lambda i: (i, 0)),
            pl.BlockSpec((BT, L), lambda i: (i, 0)),
            pl.BlockSpec((L, BT), lambda i: (0, i)),
            pl.BlockSpec((L, BT), lambda i: (0, i)),
        ],
        out_specs=pl.BlockSpec((BT, L), lambda i: (i, 0)),
        out_shape=jax.ShapeDtypeStruct((B, L), jnp.float32),
    )(m1, s1, m2_t, s2_t)


def _final_log_body(m_ref, s_ref, o_ref):
    m = m_ref[...].T
    s = s_ref[...].T
    o_ref[...] = m + jnp.log(s)


def kernel(z, means, logvars, w):
    B, L = z.shape
    K = means.shape[0]
    mu_t = means.T
    lv_t = logvars.T
    w2 = w.reshape(1, K)

    a_t, b_t, c_t = pl.pallas_call(
        _prep_body,
        out_shape=[jax.ShapeDtypeStruct((L, K), jnp.float32)] * 3,
    )(mu_t, lv_t, w2)

    if K_SC == 0:
        (out,) = _run_tc(z, a_t, b_t, c_t, B, L, final=True)
        return out
    m2, s2 = _run_sc(z, a_t, b_t, c_t, B, L)
    if K_TC == 0:
        BT = 512
        return pl.pallas_call(
            _final_log_body,
            grid=(B // BT,),
            in_specs=[pl.BlockSpec((L, BT), lambda i: (0, i))] * 2,
            out_specs=pl.BlockSpec((BT, L), lambda i: (i, 0)),
            out_shape=jax.ShapeDtypeStruct((B, L), jnp.float32),
        )(m2, s2)
    m1, s1 = _run_tc(z, a_t, b_t, c_t, B, L, final=False)
    return _run_merge(m1, s1, m2, s2, B, L)
